# baseline (speedup 1.0000x reference)
import jax
import jax.numpy as jnp
from jax.experimental import pallas as pl
from jax.experimental.pallas import tpu as pltpu

L = 64
N = 32
NHID = 1024
Q = 64
B = 16
TOPK = 8
NCHUNK = 8
SCALE = 1.0 / 32.0


def _attn_kernel(q_ref, k_ref, att_ref, idx_ref, scratch_ref):
    j = pl.program_id(1)
    q = q_ref[0]
    k = k_ref[:, 0].reshape(NCHUNK * L, NHID)
    s = jax.lax.dot_general(
        q, k, (((1,), (1,)), ((), ())),
        preferred_element_type=jnp.float32,
        precision=jax.lax.Precision.DEFAULT,
    )
    rows = [jnp.max(s[:, i * L:(i + 1) * L], axis=1) for i in range(NCHUNK)]
    scratch_ref[pl.ds(j * NCHUNK, NCHUNK), :] = jnp.stack(rows, axis=0)

    @pl.when(j == pl.num_programs(1) - 1)
    def _():
        logits = scratch_ref[...] * SCALE
        m = jnp.max(logits, axis=0, keepdims=True)
        e = jnp.exp(logits - m)
        att = e / jnp.sum(e, axis=0, keepdims=True)
        att_ref[0] = att
        iota = jax.lax.broadcasted_iota(jnp.int32, (N, Q), 0)
        vals = att
        for kk in range(TOPK):
            cur = jnp.max(vals, axis=0, keepdims=True)
            idx = jnp.min(jnp.where(vals >= cur, iota, N), axis=0)
            idx_ref[0, kk, :] = idx
            vals = jnp.where(iota == idx[None, :], -jnp.inf, vals)


def kernel(query, keys):
    q_t = jnp.transpose(query, (2, 0, 1))
    k_r = keys.reshape(N, B, L, NHID)
    att_bnq, idx_bkq = pl.pallas_call(
        _attn_kernel,
        grid=(B, N // NCHUNK),
        in_specs=[
            pl.BlockSpec((1, Q, NHID), lambda b, j: (b, 0, 0)),
            pl.BlockSpec((NCHUNK, 1, L, NHID), lambda b, j: (j, b, 0, 0)),
        ],
        out_specs=[
            pl.BlockSpec((1, N, Q), lambda b, j: (b, 0, 0)),
            pl.BlockSpec((1, TOPK, Q), lambda b, j: (b, 0, 0)),
        ],
        out_shape=[
            jax.ShapeDtypeStruct((B, N, Q), jnp.float32),
            jax.ShapeDtypeStruct((B, TOPK, Q), jnp.int32),
        ],
        scratch_shapes=[pltpu.VMEM((N, Q), jnp.float32)],
    )(q_t, k_r)
    attention = jnp.transpose(att_bnq, (2, 0, 1))
    topk_indices = jnp.transpose(idx_bkq, (1, 2, 0))
    return (attention, topk_indices)


# speedup vs baseline: 1.4273x; 1.4273x over previous
import jax
import jax.numpy as jnp
from jax.experimental import pallas as pl
from jax.experimental.pallas import tpu as pltpu

L = 64
N = 32
NHID = 1024
Q = 64
B = 16
TOPK = 8
BO = 8
NOCT = B // BO
SCALE = 1.0 / 32.0


def _attn_kernel(k_ref, qt_ref, att_ref, idx_ref, smax_ref):
    l = pl.program_id(1)
    a = k_ref[:, 0].reshape(N * BO, NHID)
    s = jax.lax.dot_general(
        a, qt_ref[...], (((1,), (0,)), ((), ())),
        preferred_element_type=jnp.float32,
        precision=jax.lax.Precision.DEFAULT,
    )

    @pl.when(l == 0)
    def _():
        smax_ref[...] = s

    @pl.when(l > 0)
    def _():
        smax_ref[...] = jnp.maximum(smax_ref[...], s)

    @pl.when(l == pl.num_programs(1) - 1)
    def _():
        sm3 = smax_ref[...].reshape(N, BO, BO * Q)
        atts, idxs = [], []
        iota = jax.lax.broadcasted_iota(jnp.int32, (N, Q), 0)
        for bo in range(BO):
            logits = sm3[:, bo, bo * Q:(bo + 1) * Q] * SCALE
            m = jnp.max(logits, axis=0, keepdims=True)
            e = jnp.exp(logits - m)
            att = e / jnp.sum(e, axis=0, keepdims=True)
            atts.append(att)
            vals = att
            rows = []
            for _ in range(TOPK):
                cur = jnp.max(vals, axis=0, keepdims=True)
                idx = jnp.min(jnp.where(vals >= cur, iota, N), axis=0)
                rows.append(idx)
                vals = jnp.where(iota == idx[None, :], -jnp.inf, vals)
            idxs.append(jnp.stack(rows, axis=0))
        att_ref[...] = jnp.stack(atts, axis=0)
        idx_ref[...] = jnp.stack(idxs, axis=0)


def kernel(query, keys):
    k4 = keys.reshape(N, NOCT, BO, L * NHID)
    qt = jnp.transpose(query, (1, 2, 0)).reshape(NHID, B * Q)
    att_bnq, idx_bkq = pl.pallas_call(
        _attn_kernel,
        grid=(NOCT, L),
        in_specs=[
            pl.BlockSpec((N, 1, BO, NHID), lambda o, l: (0, o, 0, l)),
            pl.BlockSpec((NHID, BO * Q), lambda o, l: (0, o)),
        ],
        out_specs=[
            pl.BlockSpec((BO, N, Q), lambda o, l: (o, 0, 0)),
            pl.BlockSpec((BO, TOPK, Q), lambda o, l: (o, 0, 0)),
        ],
        out_shape=[
            jax.ShapeDtypeStruct((B, N, Q), jnp.float32),
            jax.ShapeDtypeStruct((B, TOPK, Q), jnp.int32),
        ],
        scratch_shapes=[pltpu.VMEM((N * BO, BO * Q), jnp.float32)],
    )(k4, qt)
    attention = jnp.transpose(att_bnq, (2, 0, 1))
    topk_indices = jnp.transpose(idx_bkq, (1, 2, 0))
    return (attention, topk_indices)


# speedup vs baseline: 2.8855x; 2.0217x over previous
import jax
import jax.numpy as jnp
from jax.experimental import pallas as pl
from jax.experimental.pallas import tpu as pltpu

L = 64
N = 32
NHID = 1024
Q = 64
B = 16
TOPK = 8
BO = 8
NOCT = B // BO
LCH = 8
SCALE = 1.0 / 32.0


def _attn_kernel(k_ref, qt_ref, att_ref, idx_ref, smax_ref):
    l = pl.program_id(1)
    a = k_ref[:, 0].reshape(N * BO, LCH * NHID).astype(jnp.bfloat16)
    qt = qt_ref[...]
    parts = [
        jax.lax.dot_general(
            a[:, i * NHID:(i + 1) * NHID], qt, (((1,), (0,)), ((), ())),
            preferred_element_type=jnp.float32,
            precision=jax.lax.Precision.DEFAULT,
        )
        for i in range(LCH)
    ]
    s = parts[0]
    for p in parts[1:]:
        s = jnp.maximum(s, p)

    @pl.when(l == 0)
    def _():
        smax_ref[...] = s

    @pl.when(l > 0)
    def _():
        smax_ref[...] = jnp.maximum(smax_ref[...], s)

    @pl.when(l == pl.num_programs(1) - 1)
    def _():
        sm3 = smax_ref[...].reshape(N, BO, BO * Q)
        atts, idxs = [], []
        iota = jax.lax.broadcasted_iota(jnp.int32, (N, Q), 0)
        for bo in range(BO):
            logits = sm3[:, bo, bo * Q:(bo + 1) * Q] * SCALE
            m = jnp.max(logits, axis=0, keepdims=True)
            e = jnp.exp(logits - m)
            att = e / jnp.sum(e, axis=0, keepdims=True)
            atts.append(att)
            vals = att
            rows = []
            for _ in range(TOPK):
                cur = jnp.max(vals, axis=0, keepdims=True)
                idx = jnp.min(jnp.where(vals >= cur, iota, N), axis=0)
                rows.append(idx)
                vals = jnp.where(iota == idx[None, :], -jnp.inf, vals)
            idxs.append(jnp.stack(rows, axis=0))
        att_ref[...] = jnp.stack(atts, axis=0)
        idx_ref[...] = jnp.stack(idxs, axis=0)


def kernel(query, keys):
    k4 = keys.reshape(N, NOCT, BO, L * NHID)
    qt = jnp.transpose(query, (1, 2, 0)).reshape(NHID, B * Q)
    qt = qt.astype(jnp.bfloat16)
    att_bnq, idx_bkq = pl.pallas_call(
        _attn_kernel,
        grid=(NOCT, L // LCH),
        in_specs=[
            pl.BlockSpec((N, 1, BO, LCH * NHID), lambda o, l: (0, o, 0, l)),
            pl.BlockSpec((NHID, BO * Q), lambda o, l: (0, o)),
        ],
        out_specs=[
            pl.BlockSpec((BO, N, Q), lambda o, l: (o, 0, 0)),
            pl.BlockSpec((BO, TOPK, Q), lambda o, l: (o, 0, 0)),
        ],
        out_shape=[
            jax.ShapeDtypeStruct((B, N, Q), jnp.float32),
            jax.ShapeDtypeStruct((B, TOPK, Q), jnp.int32),
        ],
        scratch_shapes=[pltpu.VMEM((N * BO, BO * Q), jnp.float32)],
    )(k4, qt)
    attention = jnp.transpose(att_bnq, (2, 0, 1))
    topk_indices = jnp.transpose(idx_bkq, (1, 2, 0))
    return (attention, topk_indices)


# speedup vs baseline: 3.0788x; 1.0670x over previous
import jax
import jax.numpy as jnp
from jax.experimental import pallas as pl
from jax.experimental.pallas import tpu as pltpu

L = 64
N = 32
NHID = 1024
Q = 64
B = 16
TOPK = 8
BO = 8
NOCT = B // BO
LCH = 16
SCALE = 1.0 / 32.0


def _attn_kernel(k_ref, qt_ref, att_ref, idx_ref, smax_ref):
    l = pl.program_id(1)
    a = k_ref[:, 0].reshape(N * BO, LCH * NHID).astype(jnp.bfloat16)
    qt = qt_ref[...]
    parts = [
        jax.lax.dot_general(
            a[:, i * NHID:(i + 1) * NHID], qt, (((1,), (0,)), ((), ())),
            preferred_element_type=jnp.float32,
            precision=jax.lax.Precision.DEFAULT,
        )
        for i in range(LCH)
    ]
    s = parts[0]
    for p in parts[1:]:
        s = jnp.maximum(s, p)

    @pl.when(l == 0)
    def _():
        smax_ref[...] = s

    @pl.when(l > 0)
    def _():
        smax_ref[...] = jnp.maximum(smax_ref[...], s)

    @pl.when(l == pl.num_programs(1) - 1)
    def _():
        sm3 = smax_ref[...].reshape(N, BO, BO * Q)
        atts, idxs = [], []
        iota = jax.lax.broadcasted_iota(jnp.int32, (N, Q), 0)
        for bo in range(BO):
            logits = sm3[:, bo, bo * Q:(bo + 1) * Q] * SCALE
            m = jnp.max(logits, axis=0, keepdims=True)
            e = jnp.exp(logits - m)
            att = e / jnp.sum(e, axis=0, keepdims=True)
            atts.append(att)
            vals = att
            rows = []
            for _ in range(TOPK):
                cur = jnp.max(vals, axis=0, keepdims=True)
                idx = jnp.min(jnp.where(vals >= cur, iota, N), axis=0)
                rows.append(idx)
                vals = jnp.where(iota == idx[None, :], -jnp.inf, vals)
            idxs.append(jnp.stack(rows, axis=0))
        att_ref[...] = jnp.stack(atts, axis=0)
        idx_ref[...] = jnp.stack(idxs, axis=0)


def kernel(query, keys):
    k4 = keys.reshape(N, NOCT, BO, L * NHID)
    qt = jnp.transpose(query, (1, 2, 0)).reshape(NHID, B * Q)
    qt = qt.astype(jnp.bfloat16)
    att_bnq, idx_bkq = pl.pallas_call(
        _attn_kernel,
        grid=(NOCT, L // LCH),
        in_specs=[
            pl.BlockSpec((N, 1, BO, LCH * NHID), lambda o, l: (0, o, 0, l)),
            pl.BlockSpec((NHID, BO * Q), lambda o, l: (0, o)),
        ],
        out_specs=[
            pl.BlockSpec((BO, N, Q), lambda o, l: (o, 0, 0)),
            pl.BlockSpec((BO, TOPK, Q), lambda o, l: (o, 0, 0)),
        ],
        out_shape=[
            jax.ShapeDtypeStruct((B, N, Q), jnp.float32),
            jax.ShapeDtypeStruct((B, TOPK, Q), jnp.int32),
        ],
        scratch_shapes=[pltpu.VMEM((N * BO, BO * Q), jnp.float32)],
    )(k4, qt)
    attention = jnp.transpose(att_bnq, (2, 0, 1))
    topk_indices = jnp.transpose(idx_bkq, (1, 2, 0))
    return (attention, topk_indices)
